# Initial kernel scaffold; baseline (speedup 1.0000x reference)
import jax
import jax.numpy as jnp
from jax.experimental import pallas as pl


def kernel(x, table):
    raise NotImplementedError("write your pallas kernel here")


# speedup vs baseline: 5.1602x; 5.1602x over previous
import functools

import jax
import jax.numpy as jnp
from jax import lax
from jax.experimental import pallas as pl
from jax.experimental.pallas import tpu as pltpu
from jax.experimental.pallas import tpu_sc as plsc

_INDEX_SIZE = 28996
_EMBED_DIM = 128
_BATCH = 4096
_SEQ = 200

_B = _BATCH * _SEQ
_NC = 2
_NS = 16
_NW = _NC * _NS
_B_PER_W = _B // _NW
_CHUNK = 128
_NCHUNK = _B_PER_W // _CHUNK


def _make_gather():
    mesh = plsc.VectorSubcoreMesh(core_axis_name="c", subcore_axis_name="s")

    @functools.partial(
        pl.kernel,
        mesh=mesh,
        out_type=jax.ShapeDtypeStruct((_B, _EMBED_DIM), jnp.float32),
        scratch_types=[
            pltpu.VMEM((_CHUNK,), jnp.int32),
            pltpu.VMEM((_CHUNK, _EMBED_DIM), jnp.float32),
            pltpu.SemaphoreType.DMA,
        ],
    )
    def gather_kernel(table_hbm, idx_hbm, out_hbm, idx_v, rows_v, sem):
        wid = lax.axis_index("s") * _NC + lax.axis_index("c")
        base = wid * _B_PER_W

        def body(i, _):
            off = base + i * _CHUNK
            pltpu.sync_copy(idx_hbm.at[pl.ds(off, _CHUNK)], idx_v)
            pltpu.async_copy(table_hbm.at[idx_v], rows_v, sem).wait()
            pltpu.sync_copy(rows_v, out_hbm.at[pl.ds(off, _CHUNK)])
            return 0

        lax.fori_loop(0, _NCHUNK, body, 0)

    return gather_kernel


_gather = _make_gather()


@jax.jit
def kernel(x, table):
    flat = x.reshape(_B)
    out = _gather(table, flat)
    return out.reshape(_BATCH, _SEQ, _EMBED_DIM)


# speedup vs baseline: 9.0787x; 1.7594x over previous
import functools

import jax
import jax.numpy as jnp
from jax import lax
from jax.experimental import pallas as pl
from jax.experimental.pallas import tpu as pltpu
from jax.experimental.pallas import tpu_sc as plsc

_INDEX_SIZE = 28996
_EMBED_DIM = 128
_BATCH = 4096
_SEQ = 200

_B = _BATCH * _SEQ
_NC = 2
_NS = 16
_NW = _NC * _NS
_B_PER_W = _B // _NW
_CHUNK = 128
_NCHUNK = _B_PER_W // _CHUNK
_NB = 4
_NG = _NCHUNK // _NB


def _make_gather():
    mesh = plsc.VectorSubcoreMesh(core_axis_name="c", subcore_axis_name="s")

    @functools.partial(
        pl.kernel,
        mesh=mesh,
        out_type=jax.ShapeDtypeStruct((_B, _EMBED_DIM), jnp.float32),
        scratch_types=(
            [pltpu.VMEM((_NCHUNK, _CHUNK), jnp.int32)]
            + [pltpu.VMEM((_CHUNK, _EMBED_DIM), jnp.float32) for _ in range(_NB)]
            + [pltpu.SemaphoreType.DMA for _ in range(2 * _NB)]
        ),
    )
    def gather_kernel(table_hbm, idx_hbm, out_hbm, idx_v, *rest):
        rows = rest[:_NB]
        sem_g = rest[_NB:2 * _NB]
        sem_w = rest[2 * _NB:]

        wid = lax.axis_index("s") * _NC + lax.axis_index("c")
        base = wid * _B_PER_W

        pltpu.sync_copy(idx_hbm.at[pl.ds(wid * _NCHUNK, _NCHUNK)], idx_v)

        def start_gather(j, b):
            pltpu.async_copy(table_hbm.at[idx_v.at[j]], rows[b], sem_g[b])

        def wait_gather(j, b):
            pltpu.make_async_copy(table_hbm.at[idx_v.at[j]], rows[b], sem_g[b]).wait()

        def out_slice(j):
            return out_hbm.at[pl.ds(base + j * _CHUNK, _CHUNK)]

        def start_wb(j, b):
            pltpu.async_copy(rows[b], out_slice(j), sem_w[b])

        def wait_wb(j, b):
            pltpu.make_async_copy(rows[b], out_slice(j), sem_w[b]).wait()

        for b in range(_NB):
            start_gather(b, b)

        def body(g, _):
            j0 = g * _NB
            for b in range(_NB):
                wait_gather(j0 + b, b)
                start_wb(j0 + b, b)
            for b in range(_NB):
                wait_wb(j0 + b, b)
                start_gather(j0 + _NB + b, b)
            return 0

        lax.fori_loop(0, _NG - 1, body, 0)

        j0 = (_NG - 1) * _NB
        for b in range(_NB):
            wait_gather(j0 + b, b)
            start_wb(j0 + b, b)
        for b in range(_NB):
            wait_wb(j0 + b, b)

    return gather_kernel


_gather = _make_gather()


@jax.jit
def kernel(x, table):
    idx = x.reshape(_NW * _NCHUNK, _CHUNK)
    out = _gather(table, idx)
    return out.reshape(_BATCH, _SEQ, _EMBED_DIM)


# speedup vs baseline: 9.0918x; 1.0014x over previous
import functools

import jax
import jax.numpy as jnp
from jax import lax
from jax.experimental import pallas as pl
from jax.experimental.pallas import tpu as pltpu
from jax.experimental.pallas import tpu_sc as plsc

_INDEX_SIZE = 28996
_EMBED_DIM = 128
_BATCH = 4096
_SEQ = 200

_B = _BATCH * _SEQ
_NC = 2
_NS = 16
_NW = _NC * _NS
_B_PER_W = _B // _NW
_CHUNK = 128
_NCHUNK = _B_PER_W // _CHUNK
_NB = 5
_NG = _NCHUNK // _NB


def _make_gather():
    mesh = plsc.VectorSubcoreMesh(core_axis_name="c", subcore_axis_name="s")

    @functools.partial(
        pl.kernel,
        mesh=mesh,
        out_type=jax.ShapeDtypeStruct((_B, _EMBED_DIM), jnp.float32),
        scratch_types=(
            [pltpu.VMEM((_NCHUNK, _CHUNK), jnp.int32)]
            + [pltpu.VMEM((_CHUNK, _EMBED_DIM), jnp.float32) for _ in range(_NB)]
            + [pltpu.SemaphoreType.DMA for _ in range(2 * _NB)]
        ),
    )
    def gather_kernel(table_hbm, idx_hbm, out_hbm, idx_v, *rest):
        rows = rest[:_NB]
        sem_g = rest[_NB:2 * _NB]
        sem_w = rest[2 * _NB:]

        wid = lax.axis_index("s") * _NC + lax.axis_index("c")
        base = wid * _B_PER_W

        pltpu.sync_copy(idx_hbm.at[pl.ds(wid * _NCHUNK, _NCHUNK)], idx_v)

        def start_gather(j, b):
            pltpu.async_copy(table_hbm.at[idx_v.at[j]], rows[b], sem_g[b])

        def wait_gather(j, b):
            pltpu.make_async_copy(table_hbm.at[idx_v.at[j]], rows[b], sem_g[b]).wait()

        def out_slice(j):
            return out_hbm.at[pl.ds(base + j * _CHUNK, _CHUNK)]

        def start_wb(j, b):
            pltpu.async_copy(rows[b], out_slice(j), sem_w[b])

        def wait_wb(j, b):
            pltpu.make_async_copy(rows[b], out_slice(j), sem_w[b]).wait()

        for b in range(_NB):
            start_gather(b, b)

        def body(g, _):
            j0 = g * _NB
            for b in range(_NB):
                wait_gather(j0 + b, b)
                start_wb(j0 + b, b)
            for b in range(_NB):
                wait_wb(j0 + b, b)
                start_gather(j0 + _NB + b, b)
            return 0

        lax.fori_loop(0, _NG - 1, body, 0)

        j0 = (_NG - 1) * _NB
        for b in range(_NB):
            wait_gather(j0 + b, b)
            start_wb(j0 + b, b)
        for b in range(_NB):
            wait_wb(j0 + b, b)

    return gather_kernel


_gather = _make_gather()


@jax.jit
def kernel(x, table):
    idx = x.reshape(_NW * _NCHUNK, _CHUNK)
    out = _gather(table, idx)
    return out.reshape(_BATCH, _SEQ, _EMBED_DIM)


# speedup vs baseline: 9.1632x; 1.0078x over previous
import functools

import jax
import jax.numpy as jnp
from jax import lax
from jax.experimental import pallas as pl
from jax.experimental.pallas import tpu as pltpu
from jax.experimental.pallas import tpu_sc as plsc

_INDEX_SIZE = 28996
_EMBED_DIM = 128
_BATCH = 4096
_SEQ = 200

_B = _BATCH * _SEQ
_NC = 2
_NS = 16
_NW = _NC * _NS
_B_PER_W = _B // _NW
_CHUNK = 128
_NCHUNK = _B_PER_W // _CHUNK
_NB = 5
_NG = _NCHUNK // _NB
_K = 2


def _make_gather():
    mesh = plsc.VectorSubcoreMesh(core_axis_name="c", subcore_axis_name="s")

    @functools.partial(
        pl.kernel,
        mesh=mesh,
        out_type=jax.ShapeDtypeStruct((_B, _EMBED_DIM), jnp.float32),
        scratch_types=(
            [pltpu.VMEM((_NCHUNK, _CHUNK), jnp.int32)]
            + [pltpu.VMEM((_CHUNK, _EMBED_DIM), jnp.float32) for _ in range(_NB)]
            + [pltpu.SemaphoreType.DMA for _ in range(2 * _NB)]
        ),
    )
    def gather_kernel(table_hbm, idx_hbm, out_hbm, idx_v, *rest):
        rows = rest[:_NB]
        sem_g = rest[_NB:2 * _NB]
        sem_w = rest[2 * _NB:]

        wid = lax.axis_index("s") * _NC + lax.axis_index("c")
        base = wid * _B_PER_W

        pltpu.sync_copy(idx_hbm.at[pl.ds(wid * _NCHUNK, _NCHUNK)], idx_v)

        def start_gather(j, b):
            pltpu.async_copy(table_hbm.at[idx_v.at[j]], rows[b], sem_g[b])

        def wait_gather(j, b):
            pltpu.make_async_copy(table_hbm.at[idx_v.at[j]], rows[b], sem_g[b]).wait()

        def out_slice(j):
            return out_hbm.at[pl.ds(base + j * _CHUNK, _CHUNK)]

        def start_wb(j, b):
            pltpu.async_copy(rows[b], out_slice(j), sem_w[b])

        def wait_wb(j, b):
            pltpu.make_async_copy(rows[b], out_slice(j), sem_w[b]).wait()

        def body(g, _):
            j0 = g * _NB
            for b in range(_NB):
                j = j0 + b

                @pl.when(g > 0)
                def _():
                    wait_wb(j - _NB, b)

                start_gather(j, b)
                jw = j - _K
                bw = (b - _K) % _NB
                if b >= _K:
                    wait_gather(jw, bw)
                    start_wb(jw, bw)
                else:
                    @pl.when(g > 0)
                    def _():
                        wait_gather(jw, bw)
                        start_wb(jw, bw)
            return 0

        lax.fori_loop(0, _NG, body, 0)

        for t in range(_K):
            j = _NCHUNK - _K + t
            b = j % _NB
            wait_gather(j, b)
            start_wb(j, b)
        for t in range(_NB):
            j = _NCHUNK - _NB + t
            b = j % _NB
            wait_wb(j, b)

    return gather_kernel


_gather = _make_gather()


@jax.jit
def kernel(x, table):
    idx = x.reshape(_NW * _NCHUNK, _CHUNK)
    out = _gather(table, idx)
    return out.reshape(_BATCH, _SEQ, _EMBED_DIM)


# speedup vs baseline: 9.1692x; 1.0007x over previous
import functools

import jax
import jax.numpy as jnp
from jax import lax
from jax.experimental import pallas as pl
from jax.experimental.pallas import tpu as pltpu
from jax.experimental.pallas import tpu_sc as plsc

_INDEX_SIZE = 28996
_EMBED_DIM = 128
_BATCH = 4096
_SEQ = 200

_B = _BATCH * _SEQ
_NC = 2
_NS = 16
_NW = _NC * _NS
_B_PER_W = _B // _NW
_IDXROW = 128
_GC = 2
_ROWS = _GC * _IDXROW
_NCHUNK = _B_PER_W // _ROWS
_NIDX = _B_PER_W // _IDXROW
_NB = 3
_K = 1
_NG = 33


def _make_gather():
    mesh = plsc.VectorSubcoreMesh(core_axis_name="c", subcore_axis_name="s")

    @functools.partial(
        pl.kernel,
        mesh=mesh,
        out_type=jax.ShapeDtypeStruct((_B, _EMBED_DIM), jnp.float32),
        scratch_types=(
            [pltpu.VMEM((_NIDX, _IDXROW), jnp.int32)]
            + [pltpu.VMEM((_ROWS, _EMBED_DIM), jnp.float32) for _ in range(_NB)]
            + [pltpu.SemaphoreType.DMA for _ in range(2 * _NB)]
        ),
    )
    def gather_kernel(table_hbm, idx_hbm, out_hbm, idx_v, *rest):
        rows = rest[:_NB]
        sem_g = rest[_NB:2 * _NB]
        sem_w = rest[2 * _NB:]

        wid = lax.axis_index("s") * _NC + lax.axis_index("c")
        base = wid * _B_PER_W

        pltpu.sync_copy(idx_hbm.at[wid], idx_v)

        def start_gather(j, b):
            for k in range(_GC):
                pltpu.async_copy(
                    table_hbm.at[idx_v.at[j * _GC + k]],
                    rows[b].at[pl.ds(k * _IDXROW, _IDXROW)],
                    sem_g[b],
                )

        def wait_gather(j, b):
            for k in range(_GC):
                pltpu.make_async_copy(
                    table_hbm.at[idx_v.at[j * _GC + k]],
                    rows[b].at[pl.ds(k * _IDXROW, _IDXROW)],
                    sem_g[b],
                ).wait()

        def out_slice(j):
            return out_hbm.at[pl.ds(base + j * _ROWS, _ROWS)]

        def start_wb(j, b):
            pltpu.async_copy(rows[b], out_slice(j), sem_w[b])

        def wait_wb(j, b):
            pltpu.make_async_copy(rows[b], out_slice(j), sem_w[b]).wait()

        def body(g, _):
            j0 = g * _NB
            for b in range(_NB):
                j = j0 + b

                @pl.when(g > 0)
                def _():
                    wait_wb(j - _NB, b)

                start_gather(j, b)
                jw = j - _K
                bw = (b - _K) % _NB
                if b >= _K:
                    wait_gather(jw, bw)
                    start_wb(jw, bw)
                else:
                    @pl.when(g > 0)
                    def _():
                        wait_gather(jw, bw)
                        start_wb(jw, bw)
            return 0

        lax.fori_loop(0, _NG, body, 0)

        wait_wb(_NCHUNK - 1 - _NB, 0)
        start_gather(_NCHUNK - 1, 0)
        wait_gather(_NCHUNK - 2, 2)
        start_wb(_NCHUNK - 2, 2)
        wait_gather(_NCHUNK - 1, 0)
        start_wb(_NCHUNK - 1, 0)
        wait_wb(_NCHUNK - 3, 1)
        wait_wb(_NCHUNK - 2, 2)
        wait_wb(_NCHUNK - 1, 0)

    return gather_kernel


_gather = _make_gather()


@jax.jit
def kernel(x, table):
    idx = x.reshape(_NW, _NIDX, _IDXROW)
    out = _gather(table, idx)
    return out.reshape(_BATCH, _SEQ, _EMBED_DIM)


# speedup vs baseline: 9.1827x; 1.0015x over previous
import functools

import jax
import jax.numpy as jnp
from jax import lax
from jax.experimental import pallas as pl
from jax.experimental.pallas import tpu as pltpu
from jax.experimental.pallas import tpu_sc as plsc

_INDEX_SIZE = 28996
_EMBED_DIM = 128
_BATCH = 4096
_SEQ = 200

_B = _BATCH * _SEQ
_NC = 2
_NS = 16
_NW = _NC * _NS
_B_PER_W = _B // _NW
_IDXROW = 128
_GC = 2
_ROWS = _GC * _IDXROW
_NCHUNK = _B_PER_W // _ROWS
_NIDX = _B_PER_W // _IDXROW
_NB = 3
_K = 2
_NG = 33


def _make_gather():
    mesh = plsc.VectorSubcoreMesh(core_axis_name="c", subcore_axis_name="s")

    @functools.partial(
        pl.kernel,
        mesh=mesh,
        out_type=jax.ShapeDtypeStruct((_B, _EMBED_DIM), jnp.float32),
        scratch_types=(
            [pltpu.VMEM((_NIDX, _IDXROW), jnp.int32)]
            + [pltpu.VMEM((_ROWS, _EMBED_DIM), jnp.float32) for _ in range(_NB)]
            + [pltpu.SemaphoreType.DMA for _ in range(2 * _NB)]
        ),
    )
    def gather_kernel(table_hbm, idx_hbm, out_hbm, idx_v, *rest):
        rows = rest[:_NB]
        sem_g = rest[_NB:2 * _NB]
        sem_w = rest[2 * _NB:]

        wid = lax.axis_index("s") * _NC + lax.axis_index("c")
        base = wid * _B_PER_W

        pltpu.sync_copy(idx_hbm.at[wid], idx_v)

        def start_gather(j, b):
            for k in range(_GC):
                pltpu.async_copy(
                    table_hbm.at[idx_v.at[j * _GC + k]],
                    rows[b].at[pl.ds(k * _IDXROW, _IDXROW)],
                    sem_g[b],
                )

        def wait_gather(j, b):
            for k in range(_GC):
                pltpu.make_async_copy(
                    table_hbm.at[idx_v.at[j * _GC + k]],
                    rows[b].at[pl.ds(k * _IDXROW, _IDXROW)],
                    sem_g[b],
                ).wait()

        def out_slice(j):
            return out_hbm.at[pl.ds(base + j * _ROWS, _ROWS)]

        def start_wb(j, b):
            pltpu.async_copy(rows[b], out_slice(j), sem_w[b])

        def wait_wb(j, b):
            pltpu.make_async_copy(rows[b], out_slice(j), sem_w[b]).wait()

        def body(g, _):
            j0 = g * _NB
            for b in range(_NB):
                j = j0 + b

                @pl.when(g > 0)
                def _():
                    wait_wb(j - _NB, b)

                start_gather(j, b)
                jw = j - _K
                bw = (b - _K) % _NB
                if b >= _K:
                    wait_gather(jw, bw)
                    start_wb(jw, bw)
                else:
                    @pl.when(g > 0)
                    def _():
                        wait_gather(jw, bw)
                        start_wb(jw, bw)
            return 0

        lax.fori_loop(0, _NG, body, 0)

        wait_wb(96, 0)
        start_gather(99, 0)
        wait_gather(97, 1)
        start_wb(97, 1)
        wait_gather(98, 2)
        start_wb(98, 2)
        wait_gather(99, 0)
        start_wb(99, 0)
        wait_wb(97, 1)
        wait_wb(98, 2)
        wait_wb(99, 0)

    return gather_kernel


_gather = _make_gather()


@jax.jit
def kernel(x, table):
    idx = x.reshape(_NW, _NIDX, _IDXROW)
    out = _gather(table, idx)
    return out.reshape(_BATCH, _SEQ, _EMBED_DIM)
